# Initial kernel scaffold; baseline (speedup 1.0000x reference)
import jax
import jax.numpy as jnp
from jax.experimental import pallas as pl


def kernel(nfeats, efeats, edge_index, W_msg, b_msg, W_apply, b_apply):
    raise NotImplementedError("write your pallas kernel here")


# speedup vs baseline: 3.0145x; 3.0145x over previous
import functools

import jax
import jax.numpy as jnp
from jax import lax
from jax.experimental import pallas as pl
from jax.experimental.pallas import tpu as pltpu
from jax.experimental.pallas import tpu_sc as plsc

N_NODES = 10000
N_EDGES = 320000
D_IN = 128
D_EDGE = 16
D_OUT = 128

NC = 2
NS = 16
NW = NC * NS
CHUNK = 128
N_CHUNKS = N_EDGES // CHUNK
ZCH = 16
N_ZCH = N_NODES // ZCH


def _node_proj_body(x_ref, w_ref, b_ref, o_ref):
    o_ref[...] = jnp.dot(x_ref[...], w_ref[...],
                         preferred_element_type=jnp.float32) + b_ref[...]


def _edge_proj_body(e_ref, w_ref, o_ref):
    o_ref[...] = jnp.dot(e_ref[...], w_ref[...],
                         preferred_element_type=jnp.float32)


def _apply_body(x_ref, p_ref, w1_ref, w2_ref, b_ref, o_ref):
    hn = p_ref[0] + p_ref[1]
    acc = jnp.dot(x_ref[...], w1_ref[...], preferred_element_type=jnp.float32)
    acc += jnp.dot(hn, w2_ref[...], preferred_element_type=jnp.float32)
    o_ref[...] = jnp.maximum(acc + b_ref[...], 0.0)


def _sc_segment_body(p_hbm, eb_hbm, src_hbm, dst_hbm, out_hbm,
                     srcv, dstv, prow, ebv, zv, acc, sem):
    c = lax.axis_index("c")
    s = lax.axis_index("s")
    wid = c * NS + s

    for r in range(ZCH):
        for j in range(8):
            zv[pl.ds(r, 1), pl.ds(j * 16, 16)] = jnp.zeros((1, 16), jnp.float32)

    @pl.loop(0, (N_ZCH + NS - 1) // NS)
    def _(g):
        cidx = s + NS * g

        @pl.when(cidx < N_ZCH)
        def _():
            pltpu.sync_copy(zv, acc.at[pl.ds(cidx * ZCH, ZCH)])

    plsc.subcore_barrier()

    @pl.loop(0, (N_CHUNKS + NW - 1) // NW)
    def _(g):
        ch = wid + NW * g

        @pl.when(ch < N_CHUNKS)
        def _():
            pltpu.sync_copy(src_hbm.at[pl.ds(ch, 1)], srcv)
            pltpu.sync_copy(dst_hbm.at[pl.ds(ch, 1)], dstv)
            pltpu.sync_copy(eb_hbm.at[pl.ds(ch * CHUNK, CHUNK)], ebv)
            pltpu.async_copy(p_hbm.at[srcv.at[0]], prow, sem).wait()

            @pl.loop(0, CHUNK)
            def _(i):
                for j in range(8):
                    sl = (pl.ds(i, 1), pl.ds(j * 16, 16))
                    ebv[sl] = jnp.maximum(prow[sl] + ebv[sl], 0.0)

            pltpu.sync_copy(ebv, acc.at[dstv.at[0]], add=True)

    plsc.subcore_barrier()

    @pl.loop(0, (N_ZCH + NS - 1) // NS)
    def _(g):
        cidx = s + NS * g

        @pl.when(cidx < N_ZCH)
        def _():
            pltpu.sync_copy(acc.at[pl.ds(cidx * ZCH, ZCH)], zv)
            pltpu.sync_copy(zv, out_hbm.at[c, pl.ds(cidx * ZCH, ZCH)])


@jax.jit
def kernel(nfeats, efeats, edge_index, W_msg, b_msg, W_apply, b_apply):
    edge_index = edge_index.astype(jnp.int32)
    src = edge_index[0].reshape(N_CHUNKS, CHUNK)
    dst = edge_index[1].reshape(N_CHUNKS, CHUNK)
    b_msg2 = b_msg.reshape(1, D_OUT)
    b_apply2 = b_apply.reshape(1, D_OUT)

    p = pl.pallas_call(
        _node_proj_body,
        out_shape=jax.ShapeDtypeStruct((N_NODES, D_OUT), jnp.float32),
    )(nfeats, W_msg[:D_IN], b_msg2)

    EBLK = 4000
    eb = pl.pallas_call(
        _edge_proj_body,
        grid=(N_EDGES // EBLK,),
        in_specs=[
            pl.BlockSpec((EBLK, D_EDGE), lambda i: (i, 0)),
            pl.BlockSpec((D_EDGE, D_OUT), lambda i: (0, 0)),
        ],
        out_specs=pl.BlockSpec((EBLK, D_OUT), lambda i: (i, 0)),
        out_shape=jax.ShapeDtypeStruct((N_EDGES, D_OUT), jnp.float32),
    )(efeats, W_msg[D_IN:])

    mesh = plsc.VectorSubcoreMesh(core_axis_name="c", subcore_axis_name="s")
    sc_fn = pl.kernel(
        _sc_segment_body,
        out_type=jax.ShapeDtypeStruct((NC, N_NODES, D_OUT), jnp.float32),
        mesh=mesh,
        scratch_types=[
            pltpu.VMEM((1, CHUNK), jnp.int32),
            pltpu.VMEM((1, CHUNK), jnp.int32),
            pltpu.VMEM((CHUNK, D_OUT), jnp.float32),
            pltpu.VMEM((CHUNK, D_OUT), jnp.float32),
            pltpu.VMEM((ZCH, D_OUT), jnp.float32),
            pltpu.VMEM_SHARED((N_NODES, D_OUT), jnp.float32),
            pltpu.SemaphoreType.DMA,
        ],
    )
    partials = sc_fn(p, eb, src, dst)

    ABLK = 1000
    h = pl.pallas_call(
        _apply_body,
        grid=(N_NODES // ABLK,),
        in_specs=[
            pl.BlockSpec((ABLK, D_IN), lambda i: (i, 0)),
            pl.BlockSpec((NC, ABLK, D_OUT), lambda i: (0, i, 0)),
            pl.BlockSpec((D_IN, D_OUT), lambda i: (0, 0)),
            pl.BlockSpec((D_OUT, D_OUT), lambda i: (0, 0)),
            pl.BlockSpec((1, D_OUT), lambda i: (0, 0)),
        ],
        out_specs=pl.BlockSpec((ABLK, D_OUT), lambda i: (i, 0)),
        out_shape=jax.ShapeDtypeStruct((N_NODES, D_OUT), jnp.float32),
    )(nfeats, partials, W_apply[:D_IN], W_apply[D_IN:], b_apply2)
    return h


# speedup vs baseline: 4.6729x; 1.5502x over previous
import functools

import jax
import jax.numpy as jnp
from jax import lax
from jax.experimental import pallas as pl
from jax.experimental.pallas import tpu as pltpu
from jax.experimental.pallas import tpu_sc as plsc

N_NODES = 10000
N_EDGES = 320000
D_IN = 128
D_EDGE = 16
D_OUT = 128

NC = 2
NS = 16
NW = NC * NS
CHUNK = 64
N_CHUNKS = N_EDGES // CHUNK
NG_MAX = (N_CHUNKS + NW - 1) // NW
ZCH = 16
N_ZCH = N_NODES // ZCH


def _node_proj_body(x_ref, w_ref, b_ref, o_ref):
    o_ref[...] = jnp.dot(x_ref[...], w_ref[...],
                         preferred_element_type=jnp.float32) + b_ref[...]


def _edge_proj_body(e_ref, w_ref, o_ref):
    o_ref[...] = jnp.dot(e_ref[...], w_ref[...],
                         preferred_element_type=jnp.float32)


def _apply_body(x_ref, p_ref, w1_ref, w2_ref, b_ref, o_ref):
    hn = p_ref[0] + p_ref[1]
    acc = jnp.dot(x_ref[...], w1_ref[...], preferred_element_type=jnp.float32)
    acc += jnp.dot(hn, w2_ref[...], preferred_element_type=jnp.float32)
    o_ref[...] = jnp.maximum(acc + b_ref[...], 0.0)


def _sc_segment_body(p_hbm, eb_hbm, src_hbm, dst_hbm, out_hbm,
                     srcv, dstv, prow, ebv, zv,
                     isems, dsems, gsems, ssems, osem, acc):
    c = lax.axis_index("c")
    s = lax.axis_index("s")
    wid = c * NS + s

    for r in range(ZCH):
        for j in range(8):
            zv[pl.ds(r, 1), pl.ds(j * 16, 16)] = jnp.zeros((1, 16), jnp.float32)

    n_zero = (N_ZCH + NS - 1) // NS

    @pl.loop(0, n_zero)
    def _(g):
        cidx = s + NS * g

        @pl.when(cidx < N_ZCH)
        def _():
            pltpu.async_copy(zv, acc.at[pl.ds(cidx * ZCH, ZCH)], osem)

    @pl.loop(0, n_zero)
    def _(g):
        cidx = s + NS * g

        @pl.when(cidx < N_ZCH)
        def _():
            pltpu.make_async_copy(zv, acc.at[pl.ds(cidx * ZCH, ZCH)],
                                  osem).wait()

    plsc.subcore_barrier()

    def stage_a(g, i, checked=True):
        ch = wid + NW * g

        def body():
            pltpu.async_copy(src_hbm.at[pl.ds(ch, 1)], srcv[i], isems[i])
            pltpu.async_copy(dst_hbm.at[pl.ds(ch, 1)], dstv[i], isems[i])

        if checked:
            pl.when(ch < N_CHUNKS)(body)
        else:
            body()

    def stage_b(g, i, d, drain, checked=True):
        ch = wid + NW * g

        def body():
            pltpu.make_async_copy(src_hbm.at[pl.ds(ch, 1)], srcv[i],
                                  isems[i]).wait()
            pltpu.make_async_copy(dst_hbm.at[pl.ds(ch, 1)], dstv[i],
                                  isems[i]).wait()
            if drain:
                pltpu.make_async_copy(ebv[d], acc.at[dstv[(i + 2) % 4].at[0]],
                                      ssems[d]).wait()
            pltpu.async_copy(eb_hbm.at[pl.ds(ch * CHUNK, CHUNK)], ebv[d],
                             dsems[d])
            pltpu.async_copy(p_hbm.at[srcv[i].at[0]], prow[d], gsems[d])

        if checked:
            pl.when(ch < N_CHUNKS)(body)
        else:
            body()

    def stage_c(g, i, d, checked=True):
        ch = wid + NW * g

        def body():
            pltpu.make_async_copy(eb_hbm.at[pl.ds(ch * CHUNK, CHUNK)], ebv[d],
                                  dsems[d]).wait()
            pltpu.make_async_copy(p_hbm.at[srcv[i].at[0]], prow[d],
                                  gsems[d]).wait()

            @pl.loop(0, CHUNK)
            def _(r):
                for j in range(8):
                    sl = (pl.ds(r, 1), pl.ds(j * 16, 16))
                    ebv[d][sl] = jnp.maximum(prow[d][sl] + ebv[d][sl], 0.0)

            pltpu.async_copy(ebv[d], acc.at[dstv[i].at[0]], ssems[d],
                             add=True)

        if checked:
            pl.when(ch < N_CHUNKS)(body)
        else:
            body()

    stage_a(0, 0, checked=False)
    stage_a(1, 1, checked=False)
    stage_b(0, 0, 0, drain=False, checked=False)
    stage_a(2, 2, checked=False)
    for b in range(4):
        g = b
        stage_b(g + 1, (b + 1) % 4, (b + 1) % 2, drain=(g >= 1),
                checked=False)
        stage_c(g, b % 4, b % 2, checked=False)
        stage_a(g + 3, (b + 3) % 4, checked=False)

    @pl.loop(4, ((NG_MAX + 3) // 4) * 4, step=4)
    def _(t):
        for b in range(4):
            g = t + b
            stage_b(g + 1, (b + 1) % 4, (b + 1) % 2, drain=True)
            stage_c(g, b % 4, b % 2)
            stage_a(g + 3, (b + 3) % 4)

    for g in range(NG_MAX - 3, NG_MAX):
        ch = wid + NW * g

        @pl.when(jnp.logical_and(ch < N_CHUNKS, ch + 2 * NW >= N_CHUNKS))
        def _():
            pltpu.make_async_copy(ebv[g % 2], acc.at[dstv[g % 4].at[0]],
                                  ssems[g % 2]).wait()

    plsc.subcore_barrier()

    @pl.loop(0, n_zero)
    def _(g):
        cidx = s + NS * g

        @pl.when(cidx < N_ZCH)
        def _():
            pltpu.async_copy(acc.at[pl.ds(cidx * ZCH, ZCH)],
                             out_hbm.at[c, pl.ds(cidx * ZCH, ZCH)], osem)

    @pl.loop(0, n_zero)
    def _(g):
        cidx = s + NS * g

        @pl.when(cidx < N_ZCH)
        def _():
            pltpu.make_async_copy(acc.at[pl.ds(cidx * ZCH, ZCH)],
                                  out_hbm.at[c, pl.ds(cidx * ZCH, ZCH)],
                                  osem).wait()


@jax.jit
def kernel(nfeats, efeats, edge_index, W_msg, b_msg, W_apply, b_apply):
    edge_index = edge_index.astype(jnp.int32)
    src = edge_index[0].reshape(N_CHUNKS, CHUNK)
    dst = edge_index[1].reshape(N_CHUNKS, CHUNK)
    b_msg2 = b_msg.reshape(1, D_OUT)
    b_apply2 = b_apply.reshape(1, D_OUT)

    p = pl.pallas_call(
        _node_proj_body,
        out_shape=jax.ShapeDtypeStruct((N_NODES, D_OUT), jnp.float32),
    )(nfeats, W_msg[:D_IN], b_msg2)

    EBLK = 4000
    eb = pl.pallas_call(
        _edge_proj_body,
        grid=(N_EDGES // EBLK,),
        in_specs=[
            pl.BlockSpec((EBLK, D_EDGE), lambda i: (i, 0)),
            pl.BlockSpec((D_EDGE, D_OUT), lambda i: (0, 0)),
        ],
        out_specs=pl.BlockSpec((EBLK, D_OUT), lambda i: (i, 0)),
        out_shape=jax.ShapeDtypeStruct((N_EDGES, D_OUT), jnp.float32),
    )(efeats, W_msg[D_IN:])

    mesh = plsc.VectorSubcoreMesh(core_axis_name="c", subcore_axis_name="s")
    sc_fn = pl.kernel(
        _sc_segment_body,
        out_type=jax.ShapeDtypeStruct((NC, N_NODES, D_OUT), jnp.float32),
        mesh=mesh,
        scratch_types=[
            [pltpu.VMEM((1, CHUNK), jnp.int32) for _ in range(4)],
            [pltpu.VMEM((1, CHUNK), jnp.int32) for _ in range(4)],
            [pltpu.VMEM((CHUNK, D_OUT), jnp.float32) for _ in range(2)],
            [pltpu.VMEM((CHUNK, D_OUT), jnp.float32) for _ in range(2)],
            pltpu.VMEM((ZCH, D_OUT), jnp.float32),
            [pltpu.SemaphoreType.DMA for _ in range(4)],
            [pltpu.SemaphoreType.DMA for _ in range(2)],
            [pltpu.SemaphoreType.DMA for _ in range(2)],
            [pltpu.SemaphoreType.DMA for _ in range(2)],
            pltpu.SemaphoreType.DMA,
            pltpu.VMEM_SHARED((N_NODES, D_OUT), jnp.float32),
        ],
    )
    partials = sc_fn(p, eb, src, dst)

    ABLK = 1000
    h = pl.pallas_call(
        _apply_body,
        grid=(N_NODES // ABLK,),
        in_specs=[
            pl.BlockSpec((ABLK, D_IN), lambda i: (i, 0)),
            pl.BlockSpec((NC, ABLK, D_OUT), lambda i: (0, i, 0)),
            pl.BlockSpec((D_IN, D_OUT), lambda i: (0, 0)),
            pl.BlockSpec((D_OUT, D_OUT), lambda i: (0, 0)),
            pl.BlockSpec((1, D_OUT), lambda i: (0, 0)),
        ],
        out_specs=pl.BlockSpec((ABLK, D_OUT), lambda i: (i, 0)),
        out_shape=jax.ShapeDtypeStruct((N_NODES, D_OUT), jnp.float32),
    )(nfeats, partials, W_apply[:D_IN], W_apply[D_IN:], b_apply2)
    return h
